# Initial kernel scaffold; baseline (speedup 1.0000x reference)
import functools

import jax
import jax.numpy as jnp
from jax import lax
from jax.experimental import pallas as pl
from jax.experimental.pallas import tpu as pltpu
from jax.experimental.pallas import tpu_sc as plsc

N = 10000
E = 320000
R = 8

NC = 2
NS = 16
NW = NC * NS

K = 128
EPW = 10240
STEPS = EPW // K
E_PAD = EPW * NW
NP = 10016
ROWS = NP // NS


def _mm1_kernel(x_ref, v_ref, comb_ref, loopw_ref, y_ref, loop_ref):
    w = jnp.sum(comb_ref[...][:, :, None, None] * v_ref[...][None], axis=1)
    x = x_ref[...]
    for r in range(R):
        y_ref[r] = jnp.dot(x, w[r], preferred_element_type=jnp.float32)
    loop_ref[...] = jnp.dot(x, loopw_ref[...], preferred_element_type=jnp.float32)


def _mm2_kernel(p_ref, loop1_ref, b1_ref, v_ref, comb_ref, loopw_ref,
                y_ref, loop_ref):
    h = p_ref[0] + p_ref[1] + loop1_ref[...] + b1_ref[...]
    h = jnp.maximum(h, 0.0)
    w = jnp.sum(comb_ref[...][:, :, None, None] * v_ref[...][None], axis=1)
    for r in range(R):
        y_ref[r] = jnp.dot(h, w[r], preferred_element_type=jnp.float32)
    loop_ref[...] = jnp.dot(h, loopw_ref[...], preferred_element_type=jnp.float32)


def _final_kernel(p_ref, loop2_ref, b2_ref, out_ref):
    i = pl.program_id(0)

    @pl.when(i == 0)
    def _():
        out_ref[...] = jnp.zeros_like(out_ref)

    h = p_ref[0] + p_ref[1] + loop2_ref[...] + b2_ref[...]
    h = jnp.maximum(h, 0.0)
    out_ref[...] += jnp.sum(h, axis=0, keepdims=True) * (1.0 / N)


def _make_sc_scatter(d):
    mesh = plsc.VectorSubcoreMesh(core_axis_name="c", subcore_axis_name="s")

    @functools.partial(
        pl.kernel,
        mesh=mesh,
        out_type=jax.ShapeDtypeStruct((NC, NP, d), jnp.float32),
        scratch_types=[
            pltpu.VMEM((STEPS, K), jnp.int32),
            pltpu.VMEM((STEPS, K), jnp.int32),
            pltpu.VMEM((K, d), jnp.float32),
            pltpu.VMEM_SHARED((NP, d), jnp.float32),
            pltpu.SemaphoreType.DMA,
        ],
    )
    def sc_kernel(gidx_hbm, didx_hbm, zeros_hbm, table_hbm, out_hbm,
                  gidx_v, didx_v, rows_v, acc_sh, sem):
        cid = lax.axis_index("c")
        sid = lax.axis_index("s")
        wid = sid * NC + cid

        pltpu.sync_copy(zeros_hbm.at[pl.ds(sid * ROWS, ROWS)],
                        acc_sh.at[pl.ds(sid * ROWS, ROWS)])
        pltpu.sync_copy(gidx_hbm.at[wid], gidx_v)
        pltpu.sync_copy(didx_hbm.at[wid], didx_v)
        plsc.subcore_barrier()

        def body(j, carry):
            pltpu.async_copy(table_hbm.at[gidx_v.at[j]], rows_v, sem).wait()
            pltpu.sync_copy(rows_v, acc_sh.at[didx_v.at[j]], add=True)
            return carry

        lax.fori_loop(0, STEPS, body, 0)
        plsc.subcore_barrier()

        pltpu.sync_copy(acc_sh.at[pl.ds(sid * ROWS, ROWS)],
                        out_hbm.at[cid, pl.ds(sid * ROWS, ROWS)])

    return sc_kernel


_sc_scatter_64 = _make_sc_scatter(64)
_sc_scatter_16 = _make_sc_scatter(16)

_BN = 400
_GRID = N // _BN


def _mm1(x, v1, comb1, loop_w1):
    h = v1.shape[-1]
    return pl.pallas_call(
        _mm1_kernel,
        grid=(_GRID,),
        in_specs=[
            pl.BlockSpec((_BN, x.shape[1]), lambda i: (i, 0)),
            pl.BlockSpec(v1.shape, lambda i: (0, 0, 0)),
            pl.BlockSpec(comb1.shape, lambda i: (0, 0)),
            pl.BlockSpec(loop_w1.shape, lambda i: (0, 0)),
        ],
        out_specs=[
            pl.BlockSpec((R, _BN, h), lambda i: (0, i, 0)),
            pl.BlockSpec((_BN, h), lambda i: (i, 0)),
        ],
        out_shape=[
            jax.ShapeDtypeStruct((R, N, h), jnp.float32),
            jax.ShapeDtypeStruct((N, h), jnp.float32),
        ],
    )(x, v1, comb1, loop_w1)


def _mm2(parts, loop1, b1, v2, comb2, loop_w2):
    h = parts.shape[-1]
    c = v2.shape[-1]
    return pl.pallas_call(
        _mm2_kernel,
        grid=(_GRID,),
        in_specs=[
            pl.BlockSpec((NC, _BN, h), lambda i: (0, i, 0)),
            pl.BlockSpec((_BN, h), lambda i: (i, 0)),
            pl.BlockSpec((1, h), lambda i: (0, 0)),
            pl.BlockSpec(v2.shape, lambda i: (0, 0, 0)),
            pl.BlockSpec(comb2.shape, lambda i: (0, 0)),
            pl.BlockSpec(loop_w2.shape, lambda i: (0, 0)),
        ],
        out_specs=[
            pl.BlockSpec((R, _BN, c), lambda i: (0, i, 0)),
            pl.BlockSpec((_BN, c), lambda i: (i, 0)),
        ],
        out_shape=[
            jax.ShapeDtypeStruct((R, N, c), jnp.float32),
            jax.ShapeDtypeStruct((N, c), jnp.float32),
        ],
    )(parts, loop1, b1, v2, comb2, loop_w2)


def _final(parts, loop2, b2):
    c = parts.shape[-1]
    return pl.pallas_call(
        _final_kernel,
        grid=(_GRID,),
        in_specs=[
            pl.BlockSpec((NC, _BN, c), lambda i: (0, i, 0)),
            pl.BlockSpec((_BN, c), lambda i: (i, 0)),
            pl.BlockSpec((1, c), lambda i: (0, 0)),
        ],
        out_specs=pl.BlockSpec((1, c), lambda i: (0, 0)),
        out_shape=jax.ShapeDtypeStruct((1, c), jnp.float32),
    )(parts, loop2, b2)


def kernel(in_feat, edge_index, etypes, V1, comb1, loop_w1, b1,
           V2, comb2, loop_w2, b2):
    src = edge_index[0]
    dst = edge_index[1]
    et = etypes.reshape(-1)

    gidx = (et * N + src).astype(jnp.int32)
    didx = dst.astype(jnp.int32)
    pad = E_PAD - E
    gidx = jnp.concatenate([gidx, jnp.zeros((pad,), jnp.int32)])
    didx = jnp.concatenate([didx, jnp.full((pad,), N, jnp.int32)])
    gidx = gidx.reshape(NW, STEPS, K)
    didx = didx.reshape(NW, STEPS, K)

    h = V1.shape[-1]
    c = V2.shape[-1]
    zeros_h = jnp.zeros((NP, h), jnp.float32)
    zeros_c = jnp.zeros((NP, c), jnp.float32)

    y1, loop1 = _mm1(in_feat, V1, comb1, loop_w1)
    parts1 = _sc_scatter_64(gidx, didx, zeros_h, y1.reshape(R * N, h))
    parts1 = parts1[:, :N]

    y2, loop2 = _mm2(parts1, loop1, b1.reshape(1, h), V2, comb2, loop_w2)
    parts2 = _sc_scatter_16(gidx, didx, zeros_c, y2.reshape(R * N, c))
    parts2 = parts2[:, :N]

    return _final(parts2, loop2, b2.reshape(1, c))


# speedup vs baseline: 15.0634x; 15.0634x over previous
import functools

import jax
import jax.numpy as jnp
from jax import lax
from jax.experimental import pallas as pl
from jax.experimental.pallas import tpu as pltpu
from jax.experimental.pallas import tpu_sc as plsc

N = 10000
E = 320000
R = 8

NC = 2
NS = 16
NW = NC * NS

K = 128
EPW = 10240
STEPS = EPW // K
E_PAD = EPW * NW
NP = 10112
ROWS = NP // NS


def _mm1_kernel(x_ref, v_ref, comb_ref, loopw_ref, y_ref, loop_ref):
    w = jnp.sum(comb_ref[...][:, :, None, None] * v_ref[...][None], axis=1)
    x = x_ref[...]
    for r in range(R):
        y_ref[r] = jnp.dot(x, w[r], preferred_element_type=jnp.float32)
    loop_ref[...] = jnp.dot(x, loopw_ref[...], preferred_element_type=jnp.float32)


def _mm2_kernel(p_ref, loop1_ref, b1_ref, v_ref, comb_ref, loopw_ref,
                y_ref, loop_ref):
    h = p_ref[0] + p_ref[1] + loop1_ref[...] + b1_ref[...]
    h = jnp.maximum(h, 0.0)
    w = jnp.sum(comb_ref[...][:, :, None, None] * v_ref[...][None], axis=1)
    for r in range(R):
        y_ref[r] = jnp.dot(h, w[r], preferred_element_type=jnp.float32)
    loop_ref[...] = jnp.dot(h, loopw_ref[...], preferred_element_type=jnp.float32)


def _final_kernel(p_ref, loop2_ref, b2_ref, out_ref):
    i = pl.program_id(0)

    @pl.when(i == 0)
    def _():
        out_ref[...] = jnp.zeros_like(out_ref)

    h = p_ref[0] + p_ref[1] + loop2_ref[...] + b2_ref[...]
    h = jnp.maximum(h, 0.0)
    out_ref[...] += jnp.sum(h, axis=0, keepdims=True) * (1.0 / N)


def _make_sc_scatter(d):
    mesh = plsc.VectorSubcoreMesh(core_axis_name="c", subcore_axis_name="s")

    @functools.partial(
        pl.kernel,
        mesh=mesh,
        out_type=jax.ShapeDtypeStruct((NC, NP, d), jnp.float32),
        scratch_types=[
            pltpu.VMEM((STEPS, K), jnp.int32),
            pltpu.VMEM((STEPS, K), jnp.int32),
            pltpu.VMEM((K, d), jnp.float32),
            pltpu.VMEM_SHARED((NP, d), jnp.float32),
            pltpu.SemaphoreType.DMA,
        ],
        compiler_params=pltpu.CompilerParams(use_tc_tiling_on_sc=False),
    )
    def sc_kernel(gidx_hbm, didx_hbm, zeros_hbm, table_hbm, out_hbm,
                  gidx_v, didx_v, rows_v, acc_sh, sem):
        cid = lax.axis_index("c")
        sid = lax.axis_index("s")
        wid = sid * NC + cid

        pltpu.sync_copy(zeros_hbm.at[pl.ds(sid * ROWS, ROWS)],
                        acc_sh.at[pl.ds(sid * ROWS, ROWS)])
        pltpu.sync_copy(gidx_hbm.at[wid], gidx_v)
        pltpu.sync_copy(didx_hbm.at[wid], didx_v)
        plsc.subcore_barrier()

        def body(j, carry):
            pltpu.async_copy(table_hbm.at[gidx_v.at[j]], rows_v, sem).wait()
            pltpu.sync_copy(rows_v, acc_sh.at[didx_v.at[j]], add=True)
            return carry

        lax.fori_loop(0, STEPS, body, 0)
        plsc.subcore_barrier()

        pltpu.sync_copy(acc_sh.at[pl.ds(sid * ROWS, ROWS)],
                        out_hbm.at[cid, pl.ds(sid * ROWS, ROWS)])

    return sc_kernel


_sc_scatter_64 = _make_sc_scatter(64)
_sc_scatter_16 = _make_sc_scatter(16)

_BN = 400
_GRID = N // _BN


def _mm1(x, v1, comb1, loop_w1):
    h = v1.shape[-1]
    return pl.pallas_call(
        _mm1_kernel,
        grid=(_GRID,),
        in_specs=[
            pl.BlockSpec((_BN, x.shape[1]), lambda i: (i, 0)),
            pl.BlockSpec(v1.shape, lambda i: (0, 0, 0)),
            pl.BlockSpec(comb1.shape, lambda i: (0, 0)),
            pl.BlockSpec(loop_w1.shape, lambda i: (0, 0)),
        ],
        out_specs=[
            pl.BlockSpec((R, _BN, h), lambda i: (0, i, 0)),
            pl.BlockSpec((_BN, h), lambda i: (i, 0)),
        ],
        out_shape=[
            jax.ShapeDtypeStruct((R, N, h), jnp.float32),
            jax.ShapeDtypeStruct((N, h), jnp.float32),
        ],
    )(x, v1, comb1, loop_w1)


def _mm2(parts, loop1, b1, v2, comb2, loop_w2):
    h = parts.shape[-1]
    c = v2.shape[-1]
    return pl.pallas_call(
        _mm2_kernel,
        grid=(_GRID,),
        in_specs=[
            pl.BlockSpec((NC, _BN, h), lambda i: (0, i, 0)),
            pl.BlockSpec((_BN, h), lambda i: (i, 0)),
            pl.BlockSpec((1, h), lambda i: (0, 0)),
            pl.BlockSpec(v2.shape, lambda i: (0, 0, 0)),
            pl.BlockSpec(comb2.shape, lambda i: (0, 0)),
            pl.BlockSpec(loop_w2.shape, lambda i: (0, 0)),
        ],
        out_specs=[
            pl.BlockSpec((R, _BN, c), lambda i: (0, i, 0)),
            pl.BlockSpec((_BN, c), lambda i: (i, 0)),
        ],
        out_shape=[
            jax.ShapeDtypeStruct((R, N, c), jnp.float32),
            jax.ShapeDtypeStruct((N, c), jnp.float32),
        ],
    )(parts, loop1, b1, v2, comb2, loop_w2)


def _final(parts, loop2, b2):
    c = parts.shape[-1]
    return pl.pallas_call(
        _final_kernel,
        grid=(_GRID,),
        in_specs=[
            pl.BlockSpec((NC, _BN, c), lambda i: (0, i, 0)),
            pl.BlockSpec((_BN, c), lambda i: (i, 0)),
            pl.BlockSpec((1, c), lambda i: (0, 0)),
        ],
        out_specs=pl.BlockSpec((1, c), lambda i: (0, 0)),
        out_shape=jax.ShapeDtypeStruct((1, c), jnp.float32),
    )(parts, loop2, b2)


def kernel(in_feat, edge_index, etypes, V1, comb1, loop_w1, b1,
           V2, comb2, loop_w2, b2):
    src = edge_index[0]
    dst = edge_index[1]
    et = etypes.reshape(-1)

    gidx = (et * N + src).astype(jnp.int32)
    didx = dst.astype(jnp.int32)
    pad = E_PAD - E
    gidx = jnp.concatenate([gidx, jnp.zeros((pad,), jnp.int32)])
    didx = jnp.concatenate([didx, jnp.full((pad,), N, jnp.int32)])
    gidx = gidx.reshape(NW, STEPS, K)
    didx = didx.reshape(NW, STEPS, K)

    h = V1.shape[-1]
    c = V2.shape[-1]
    zeros_h = jnp.zeros((NP, h), jnp.float32)
    zeros_c = jnp.zeros((NP, c), jnp.float32)

    y1, loop1 = _mm1(in_feat, V1, comb1, loop_w1)
    parts1 = _sc_scatter_64(gidx, didx, zeros_h, y1.reshape(R * N, h))
    parts1 = parts1[:, :N]

    y2, loop2 = _mm2(parts1, loop1, b1.reshape(1, h), V2, comb2, loop_w2)
    parts2 = _sc_scatter_16(gidx, didx, zeros_c, y2.reshape(R * N, c))
    parts2 = parts2[:, :N]

    return _final(parts2, loop2, b2.reshape(1, c))
